# Initial kernel scaffold; baseline (speedup 1.0000x reference)
import jax
import jax.numpy as jnp
from jax.experimental import pallas as pl


def kernel(span_reps, type_features, edge_index, edge_attr, emb_table, W, b, prelu_w):
    raise NotImplementedError("write your pallas kernel here")


# speedup vs baseline: 10.0125x; 10.0125x over previous
import functools

import jax
import jax.numpy as jnp
from jax import lax
from jax.experimental import pallas as pl
from jax.experimental.pallas import tpu as pltpu
from jax.experimental.pallas import tpu_sc as plsc

N_NODES = 10000
N_PAD = 10240
N_EDGES = 320000
E_PAD = 327680
CHUNKS = E_PAD // 128
CHUNKS_PER_TILE = CHUNKS // 32
ROWS_PER_TILE = N_PAD // 16
D_HID = 128
BLK = 1024
SENTINEL = N_NODES

_MESH = dict(core_axis_name="c", subcore_axis_name="s", num_cores=2,
             num_subcores=16)


def _deg_body(dst_hbm, out_hbm, idx_v, ones_v, zero_v, deg_sh):
    c = lax.axis_index("c")
    s = lax.axis_index("s")
    tid = c * 16 + s
    for i in range(8):
        ones_v[pl.ds(i * 16, 16)] = jnp.ones((16,), jnp.float32)

    def _zb(i, carry):
        zero_v[pl.ds(i * 16, 16)] = jnp.zeros((16,), jnp.float32)
        return carry

    lax.fori_loop(0, ROWS_PER_TILE // 16, _zb, 0)
    pltpu.sync_copy(zero_v, deg_sh.at[pl.ds(s * ROWS_PER_TILE, ROWS_PER_TILE)])
    pltpu.sync_copy(dst_hbm.at[pl.ds(tid * CHUNKS_PER_TILE, CHUNKS_PER_TILE)],
                    idx_v)
    plsc.subcore_barrier()

    def _body(t, carry):
        pltpu.sync_copy(ones_v, deg_sh.at[idx_v.at[t]], add=True)
        return carry

    lax.fori_loop(0, CHUNKS_PER_TILE, _body, 0)
    plsc.subcore_barrier()
    sl = pl.ds(s * ROWS_PER_TILE, ROWS_PER_TILE)
    pltpu.sync_copy(deg_sh.at[sl], out_hbm.at[c, sl])


@functools.cache
def _deg_call():
    return pl.kernel(
        _deg_body,
        out_type=jax.ShapeDtypeStruct((2, N_PAD), jnp.float32),
        mesh=plsc.VectorSubcoreMesh(**_MESH),
        scratch_types=[
            pltpu.VMEM((CHUNKS_PER_TILE, 128), jnp.int32),
            pltpu.VMEM((128,), jnp.float32),
            pltpu.VMEM((ROWS_PER_TILE,), jnp.float32),
            pltpu.VMEM_SHARED((N_PAD,), jnp.float32),
        ],
    )


def _agg_body(y_hbm, src_hbm, dst_hbm, zeros_hbm, out_hbm,
              sidx_v, didx_v, rows_v, acc_sh):
    c = lax.axis_index("c")
    s = lax.axis_index("s")
    tid = c * 16 + s
    sl = pl.ds(s * ROWS_PER_TILE, ROWS_PER_TILE)
    pltpu.sync_copy(zeros_hbm.at[sl], acc_sh.at[sl])
    esl = pl.ds(tid * CHUNKS_PER_TILE, CHUNKS_PER_TILE)
    pltpu.sync_copy(src_hbm.at[esl], sidx_v)
    pltpu.sync_copy(dst_hbm.at[esl], didx_v)
    plsc.subcore_barrier()

    def _body(t, carry):
        pltpu.sync_copy(y_hbm.at[sidx_v.at[t]], rows_v)
        pltpu.sync_copy(rows_v, acc_sh.at[didx_v.at[t]], add=True)
        return carry

    lax.fori_loop(0, CHUNKS_PER_TILE, _body, 0)
    plsc.subcore_barrier()
    pltpu.sync_copy(acc_sh.at[sl], out_hbm.at[c, sl])


@functools.cache
def _agg_call():
    return pl.kernel(
        _agg_body,
        out_type=jax.ShapeDtypeStruct((2, N_PAD, D_HID), jnp.float32),
        mesh=plsc.VectorSubcoreMesh(**_MESH),
        scratch_types=[
            pltpu.VMEM((CHUNKS_PER_TILE, 128), jnp.int32),
            pltpu.VMEM((CHUNKS_PER_TILE, 128), jnp.int32),
            pltpu.VMEM((128, D_HID), jnp.float32),
            pltpu.VMEM_SHARED((N_PAD, D_HID), jnp.float32),
        ],
    )


def _lin_body(span_ref, types_ref, emb_ref, wts_ref, wte_ref, dis_ref, y_ref):
    i = pl.program_id(0)
    types = types_ref[0, 0, :]
    onehot = (types[:, None]
              == lax.broadcasted_iota(jnp.int32, (1, 16), 1)).astype(jnp.float32)
    m = jnp.dot(emb_ref[...], wte_ref[...], preferred_element_type=jnp.float32)
    x = (jnp.dot(span_ref[...], wts_ref[...],
                 preferred_element_type=jnp.float32)
         + jnp.dot(onehot, m, preferred_element_type=jnp.float32))
    rowid = i * BLK + lax.broadcasted_iota(jnp.int32, (BLK, 1), 0)
    y_ref[...] = jnp.where(rowid < N_NODES, dis_ref[...] * x, 0.0)


def _linear_tc(span_p, types3, emb_table, wt_span, wt_emb, dis):
    return pl.pallas_call(
        _lin_body,
        grid=(N_PAD // BLK,),
        in_specs=[
            pl.BlockSpec((BLK, 96), lambda i: (i, 0)),
            pl.BlockSpec((1, 1, BLK), lambda i: (i, 0, 0)),
            pl.BlockSpec((16, 32), lambda i: (0, 0)),
            pl.BlockSpec((96, D_HID), lambda i: (0, 0)),
            pl.BlockSpec((32, D_HID), lambda i: (0, 0)),
            pl.BlockSpec((BLK, 1), lambda i: (i, 0)),
        ],
        out_specs=pl.BlockSpec((BLK, D_HID), lambda i: (i, 0)),
        out_shape=jax.ShapeDtypeStruct((N_PAD, D_HID), jnp.float32),
    )(span_p, types3, emb_table, wt_span, wt_emb, dis)


def _epi_body(s_ref, y_ref, dis_ref, b_ref, pw_ref, out_ref):
    ssum = s_ref[0] + s_ref[1]
    o = dis_ref[...] * (ssum + y_ref[...]) + b_ref[...]
    out_ref[...] = jnp.where(o >= 0.0, o, pw_ref[...] * o)


def _epilogue_tc(s_parts, y, dis, b2, pw2):
    return pl.pallas_call(
        _epi_body,
        grid=(N_PAD // BLK,),
        in_specs=[
            pl.BlockSpec((2, BLK, D_HID), lambda i: (0, i, 0)),
            pl.BlockSpec((BLK, D_HID), lambda i: (i, 0)),
            pl.BlockSpec((BLK, 1), lambda i: (i, 0)),
            pl.BlockSpec((1, D_HID), lambda i: (0, 0)),
            pl.BlockSpec((1, D_HID), lambda i: (0, 0)),
        ],
        out_specs=pl.BlockSpec((BLK, D_HID), lambda i: (i, 0)),
        out_shape=jax.ShapeDtypeStruct((N_PAD, D_HID), jnp.float32),
    )(s_parts, y, dis, b2, pw2)


def kernel(span_reps, type_features, edge_index, edge_attr, emb_table, W, b,
           prelu_w):
    del edge_attr
    f32 = jnp.float32
    src = edge_index[0].astype(jnp.int32)
    dst = edge_index[1].astype(jnp.int32)
    pad = jnp.full((E_PAD - N_EDGES,), SENTINEL, jnp.int32)
    src2 = jnp.concatenate([src, pad]).reshape(CHUNKS, 128)
    dst2 = jnp.concatenate([dst, pad]).reshape(CHUNKS, 128)

    deg_parts = _deg_call()(dst2)
    dis = lax.rsqrt(deg_parts[0] + deg_parts[1] + 1.0).reshape(N_PAD, 1)

    span_p = jnp.pad(span_reps, ((0, N_PAD - N_NODES), (0, 0)))
    types3 = jnp.pad(type_features.astype(jnp.int32),
                     (0, N_PAD - N_NODES)).reshape(N_PAD // BLK, 1, BLK)
    wt = W.astype(f32).T
    y = _linear_tc(span_p.astype(f32), types3, emb_table.astype(f32),
                   wt[:96], wt[96:], dis)

    zeros_rows = jnp.zeros((N_PAD, D_HID), f32)
    s_parts = _agg_call()(y, src2, dst2, zeros_rows)

    out = _epilogue_tc(s_parts, y, dis,
                       b.astype(f32).reshape(1, D_HID),
                       prelu_w.astype(f32).reshape(1, D_HID))
    return out[:N_NODES]


# speedup vs baseline: 27.3900x; 2.7356x over previous
import functools

import jax
import jax.numpy as jnp
from jax import lax
from jax.experimental import pallas as pl
from jax.experimental.pallas import tpu as pltpu
from jax.experimental.pallas import tpu_sc as plsc

N_NODES = 10000
N_PAD = 10240
N_EDGES = 320000
E_PAD = 327680
CHUNKS = E_PAD // 128
CHUNKS_PER_TILE = CHUNKS // 32
ROWS_PER_TILE = N_PAD // 16
D_HID = 128
BLK = 1024
SENTINEL = N_NODES

_MESH = dict(core_axis_name="c", subcore_axis_name="s", num_cores=2,
             num_subcores=16)


def _deg_body(dst_hbm, out_hbm, idx_v, ones_v, zero_v, deg_sh):
    c = lax.axis_index("c")
    s = lax.axis_index("s")
    tid = c * 16 + s
    for i in range(8):
        ones_v[pl.ds(i * 16, 16)] = jnp.ones((16,), jnp.float32)

    def _zb(i, carry):
        zero_v[pl.ds(i * 16, 16)] = jnp.zeros((16,), jnp.float32)
        return carry

    lax.fori_loop(0, ROWS_PER_TILE // 16, _zb, 0)
    pltpu.sync_copy(zero_v, deg_sh.at[pl.ds(s * ROWS_PER_TILE, ROWS_PER_TILE)])
    pltpu.sync_copy(dst_hbm.at[pl.ds(tid * CHUNKS_PER_TILE, CHUNKS_PER_TILE)],
                    idx_v)
    plsc.subcore_barrier()

    def _body(t, carry):
        pltpu.sync_copy(ones_v, deg_sh.at[idx_v.at[t]], add=True)
        return carry

    lax.fori_loop(0, CHUNKS_PER_TILE, _body, 0)
    plsc.subcore_barrier()
    sl = pl.ds(s * ROWS_PER_TILE, ROWS_PER_TILE)
    pltpu.sync_copy(deg_sh.at[sl], out_hbm.at[c, sl])


@functools.cache
def _deg_call():
    return pl.kernel(
        _deg_body,
        out_type=jax.ShapeDtypeStruct((2, N_PAD), jnp.float32),
        mesh=plsc.VectorSubcoreMesh(**_MESH),
        scratch_types=[
            pltpu.VMEM((CHUNKS_PER_TILE, 128), jnp.int32),
            pltpu.VMEM((128,), jnp.float32),
            pltpu.VMEM((ROWS_PER_TILE,), jnp.float32),
            pltpu.VMEM_SHARED((N_PAD,), jnp.float32),
        ],
    )


def _agg_body(y_hbm, src_hbm, dst_hbm, zeros_hbm, out_hbm,
              sidx_v, didx_v, rows_v, acc_sh):
    c = lax.axis_index("c")
    s = lax.axis_index("s")
    tid = c * 16 + s
    sl = pl.ds(s * ROWS_PER_TILE, ROWS_PER_TILE)
    pltpu.sync_copy(zeros_hbm.at[sl], acc_sh.at[sl])
    esl = pl.ds(tid * CHUNKS_PER_TILE, CHUNKS_PER_TILE)
    pltpu.sync_copy(src_hbm.at[esl], sidx_v)
    pltpu.sync_copy(dst_hbm.at[esl], didx_v)
    plsc.subcore_barrier()

    def _body(t, carry):
        pltpu.sync_copy(y_hbm.at[sidx_v.at[t]], rows_v)
        pltpu.sync_copy(rows_v, acc_sh.at[didx_v.at[t]], add=True)
        return carry

    lax.fori_loop(0, CHUNKS_PER_TILE, _body, 0)
    plsc.subcore_barrier()
    pltpu.sync_copy(acc_sh.at[sl], out_hbm.at[c, sl])


@functools.cache
def _agg_call():
    return pl.kernel(
        _agg_body,
        out_type=jax.ShapeDtypeStruct((2, N_PAD, D_HID), jnp.float32),
        mesh=plsc.VectorSubcoreMesh(**_MESH),
        scratch_types=[
            pltpu.VMEM((CHUNKS_PER_TILE, 128), jnp.int32),
            pltpu.VMEM((CHUNKS_PER_TILE, 128), jnp.int32),
            pltpu.VMEM((128, D_HID), jnp.float32),
            pltpu.VMEM_SHARED((N_PAD, D_HID), jnp.float32),
        ],
    )


def _lin_body(span_ref, types_ref, emb_ref, wts_ref, wte_ref, dis_ref, y_ref):
    i = pl.program_id(0)
    types = types_ref[0, 0, :]
    onehot = (types[:, None]
              == lax.broadcasted_iota(jnp.int32, (1, 16), 1)).astype(jnp.float32)
    m = jnp.dot(emb_ref[...], wte_ref[...], preferred_element_type=jnp.float32)
    x = (jnp.dot(span_ref[...], wts_ref[...],
                 preferred_element_type=jnp.float32)
         + jnp.dot(onehot, m, preferred_element_type=jnp.float32))
    rowid = i * BLK + lax.broadcasted_iota(jnp.int32, (BLK, 1), 0)
    y_ref[...] = jnp.where(rowid < N_NODES, dis_ref[...] * x, 0.0)


def _linear_tc(span_p, types3, emb_table, wt_span, wt_emb, dis):
    return pl.pallas_call(
        _lin_body,
        grid=(N_PAD // BLK,),
        in_specs=[
            pl.BlockSpec((BLK, 96), lambda i: (i, 0)),
            pl.BlockSpec((1, 1, BLK), lambda i: (i, 0, 0)),
            pl.BlockSpec((16, 32), lambda i: (0, 0)),
            pl.BlockSpec((96, D_HID), lambda i: (0, 0)),
            pl.BlockSpec((32, D_HID), lambda i: (0, 0)),
            pl.BlockSpec((BLK, 1), lambda i: (i, 0)),
        ],
        out_specs=pl.BlockSpec((BLK, D_HID), lambda i: (i, 0)),
        out_shape=jax.ShapeDtypeStruct((N_PAD, D_HID), jnp.float32),
    )(span_p, types3, emb_table, wt_span, wt_emb, dis)


def _epi_body(s_ref, y_ref, dis_ref, b_ref, pw_ref, out_ref):
    ssum = s_ref[0] + s_ref[1]
    o = dis_ref[...] * (ssum + y_ref[...]) + b_ref[...]
    out_ref[...] = jnp.where(o >= 0.0, o, pw_ref[...] * o)


def _epilogue_tc(s_parts, y, dis, b2, pw2):
    return pl.pallas_call(
        _epi_body,
        grid=(N_PAD // BLK,),
        in_specs=[
            pl.BlockSpec((2, BLK, D_HID), lambda i: (0, i, 0)),
            pl.BlockSpec((BLK, D_HID), lambda i: (i, 0)),
            pl.BlockSpec((BLK, 1), lambda i: (i, 0)),
            pl.BlockSpec((1, D_HID), lambda i: (0, 0)),
            pl.BlockSpec((1, D_HID), lambda i: (0, 0)),
        ],
        out_specs=pl.BlockSpec((BLK, D_HID), lambda i: (i, 0)),
        out_shape=jax.ShapeDtypeStruct((N_PAD, D_HID), jnp.float32),
    )(s_parts, y, dis, b2, pw2)


def kernel(span_reps, type_features, edge_index, edge_attr, emb_table, W, b,
           prelu_w):
    del edge_attr
    f32 = jnp.float32
    src = edge_index[0].astype(jnp.int32)
    dst = edge_index[1].astype(jnp.int32)
    npadrows = N_PAD - N_NODES
    pad = SENTINEL + (jnp.arange(E_PAD - N_EDGES, dtype=jnp.int32) % npadrows)
    src2 = jnp.concatenate([src, pad]).reshape(CHUNKS, 128)
    dst2 = jnp.concatenate([dst, pad]).reshape(CHUNKS, 128)

    deg_parts = _deg_call()(dst2)
    dis = lax.rsqrt(deg_parts[0] + deg_parts[1] + 1.0).reshape(N_PAD, 1)

    span_p = jnp.pad(span_reps, ((0, N_PAD - N_NODES), (0, 0)))
    types3 = jnp.pad(type_features.astype(jnp.int32),
                     (0, N_PAD - N_NODES)).reshape(N_PAD // BLK, 1, BLK)
    wt = W.astype(f32).T
    y = _linear_tc(span_p.astype(f32), types3, emb_table.astype(f32),
                   wt[:96], wt[96:], dis)

    zeros_rows = jnp.zeros((N_PAD, D_HID), f32)
    s_parts = _agg_call()(y, src2, dst2, zeros_rows)

    out = _epilogue_tc(s_parts, y, dis,
                       b.astype(f32).reshape(1, D_HID),
                       prelu_w.astype(f32).reshape(1, D_HID))
    return out[:N_NODES]


# speedup vs baseline: 42.3159x; 1.5449x over previous
import functools

import jax
import jax.numpy as jnp
from jax import lax
from jax.experimental import pallas as pl
from jax.experimental.pallas import tpu as pltpu
from jax.experimental.pallas import tpu_sc as plsc

N_NODES = 10000
N_PAD = 10240
N_EDGES = 320000
CW = 80
CHUNKS_PER_TILE = 128
REAL_CPT = N_EDGES // CW // 32
PAD_CPT = CHUNKS_PER_TILE - REAL_CPT
ROWS_PER_TILE = N_PAD // 16
D_HID = 128
BLK = 1024
SENTINEL = N_NODES

_MESH = dict(core_axis_name="c", subcore_axis_name="s", num_cores=2,
             num_subcores=16)


def _deg_body(ei_hbm, pad_hbm, out_hbm, idx_v, ones_v, zero_v, sem_s,
              deg_sh):
    c = lax.axis_index("c")
    s = lax.axis_index("s")
    tid = c * 16 + s
    for i in range(CW // 16):
        ones_v[pl.ds(i * 16, 16)] = jnp.ones((16,), jnp.float32)

    def _zb(i, carry):
        zero_v[pl.ds(i * 16, 16)] = jnp.zeros((16,), jnp.float32)
        return carry

    lax.fori_loop(0, ROWS_PER_TILE // 16, _zb, 0)
    pltpu.sync_copy(zero_v, deg_sh.at[pl.ds(s * ROWS_PER_TILE, ROWS_PER_TILE)])
    pltpu.sync_copy(ei_hbm.at[1, pl.ds(tid * REAL_CPT, REAL_CPT)],
                    idx_v.at[pl.ds(0, REAL_CPT)])
    pltpu.sync_copy(pad_hbm.at[pl.ds(tid * PAD_CPT, PAD_CPT)],
                    idx_v.at[pl.ds(REAL_CPT, PAD_CPT)])
    plsc.subcore_barrier()

    def _body(t, carry):
        pltpu.async_copy(ones_v, deg_sh.at[idx_v.at[t]], sem_s, add=True)

        @pl.when(t >= 8)
        def _drain_one():
            pltpu.make_async_copy(out_hbm.at[0, pl.ds(0, CW)], ones_v,
                                  sem_s).wait()

        return carry

    lax.fori_loop(0, CHUNKS_PER_TILE, _body, 0)
    for _ in range(8):
        pltpu.make_async_copy(out_hbm.at[0, pl.ds(0, CW)], ones_v,
                              sem_s).wait()
    plsc.subcore_barrier()
    sl = pl.ds(s * ROWS_PER_TILE, ROWS_PER_TILE)
    pltpu.sync_copy(deg_sh.at[sl], out_hbm.at[c, sl])


@functools.cache
def _deg_call():
    return pl.kernel(
        _deg_body,
        out_type=jax.ShapeDtypeStruct((2, N_PAD), jnp.float32),
        mesh=plsc.VectorSubcoreMesh(**_MESH),
        compiler_params=pltpu.CompilerParams(use_tc_tiling_on_sc=False),
        scratch_types=[
            pltpu.VMEM((CHUNKS_PER_TILE, CW), jnp.int32),
            pltpu.VMEM((CW,), jnp.float32),
            pltpu.VMEM((ROWS_PER_TILE,), jnp.float32),
            pltpu.SemaphoreType.DMA,
            pltpu.VMEM_SHARED((N_PAD,), jnp.float32),
        ],
    )


NB = 8
PD = 4


def _agg_body(y_hbm, ei_hbm, pad_hbm, zeros_hbm, out_hbm,
              sidx_v, didx_v, rows_v, *scr):
    gsems, sem_s, acc_sh = scr[:NB], scr[NB], scr[NB + 1]
    c = lax.axis_index("c")
    s = lax.axis_index("s")
    tid = c * 16 + s
    sl = pl.ds(s * ROWS_PER_TILE, ROWS_PER_TILE)
    pltpu.sync_copy(zeros_hbm.at[sl], acc_sh.at[sl])
    esl = pl.ds(tid * REAL_CPT, REAL_CPT)
    psl = pl.ds(tid * PAD_CPT, PAD_CPT)
    pltpu.sync_copy(ei_hbm.at[0, esl], sidx_v.at[pl.ds(0, REAL_CPT)])
    pltpu.sync_copy(pad_hbm.at[psl], sidx_v.at[pl.ds(REAL_CPT, PAD_CPT)])
    pltpu.sync_copy(ei_hbm.at[1, esl], didx_v.at[pl.ds(0, REAL_CPT)])
    pltpu.sync_copy(pad_hbm.at[psl], didx_v.at[pl.ds(REAL_CPT, PAD_CPT)])
    plsc.subcore_barrier()

    for k in range(PD):
        pltpu.async_copy(y_hbm.at[sidx_v.at[k]], rows_v.at[k], gsems[k])

    def _outer(o, carry):
        for j in range(NB):
            t = NB * o + j
            pltpu.make_async_copy(y_hbm.at[sidx_v.at[t]], rows_v.at[j],
                                  gsems[j]).wait()

            @pl.when(t >= PD)
            def _drain_scatter():
                pltpu.make_async_copy(zeros_hbm.at[pl.ds(0, CW)],
                                      rows_v.at[0], sem_s).wait()

            pltpu.async_copy(rows_v.at[j], acc_sh.at[didx_v.at[t]], sem_s,
                             add=True)
            tn = lax.rem(t + PD, CHUNKS_PER_TILE)
            jn = (j + PD) % NB
            pltpu.async_copy(y_hbm.at[sidx_v.at[tn]], rows_v.at[jn],
                             gsems[jn])
        return carry

    lax.fori_loop(0, CHUNKS_PER_TILE // NB, _outer, 0)
    for _ in range(PD):
        pltpu.make_async_copy(zeros_hbm.at[pl.ds(0, CW)], rows_v.at[0],
                              sem_s).wait()
    for k in range(PD):
        pltpu.make_async_copy(y_hbm.at[sidx_v.at[k]], rows_v.at[k],
                              gsems[k]).wait()
    plsc.subcore_barrier()
    pltpu.sync_copy(acc_sh.at[sl], out_hbm.at[c, sl])


@functools.cache
def _agg_call():
    return pl.kernel(
        _agg_body,
        out_type=jax.ShapeDtypeStruct((2, N_PAD, D_HID), jnp.bfloat16),
        mesh=plsc.VectorSubcoreMesh(**_MESH),
        compiler_params=pltpu.CompilerParams(use_tc_tiling_on_sc=False),
        scratch_types=[
            pltpu.VMEM((CHUNKS_PER_TILE, CW), jnp.int32),
            pltpu.VMEM((CHUNKS_PER_TILE, CW), jnp.int32),
            pltpu.VMEM((NB, CW, D_HID), jnp.bfloat16),
        ] + [pltpu.SemaphoreType.DMA] * (NB + 1) + [
            pltpu.VMEM_SHARED((N_PAD, D_HID), jnp.bfloat16),
        ],
    )


def _mm_body(span_ref, types_ref, emb_ref, wts_ref, wte_ref, x_ref):
    types = types_ref[0, 0, :]
    onehot = (types[:, None]
              == lax.broadcasted_iota(jnp.int32, (1, 16), 1)).astype(jnp.float32)
    m = jnp.dot(emb_ref[...], wte_ref[...], preferred_element_type=jnp.float32)
    x_ref[...] = (jnp.dot(span_ref[...], wts_ref[...],
                          preferred_element_type=jnp.float32)
                  + jnp.dot(onehot, m, preferred_element_type=jnp.float32))


def _matmul_tc(span_p, types3, emb_table, wt_span, wt_emb):
    return pl.pallas_call(
        _mm_body,
        grid=(N_PAD // BLK,),
        in_specs=[
            pl.BlockSpec((BLK, 96), lambda i: (i, 0)),
            pl.BlockSpec((1, 1, BLK), lambda i: (i, 0, 0)),
            pl.BlockSpec((16, 32), lambda i: (0, 0)),
            pl.BlockSpec((96, D_HID), lambda i: (0, 0)),
            pl.BlockSpec((32, D_HID), lambda i: (0, 0)),
        ],
        out_specs=pl.BlockSpec((BLK, D_HID), lambda i: (i, 0)),
        out_shape=jax.ShapeDtypeStruct((N_PAD, D_HID), jnp.float32),
    )(span_p, types3, emb_table, wt_span, wt_emb)


def _scale_body(x_ref, dis_ref, y_ref):
    i = pl.program_id(0)
    rowid = i * BLK + lax.broadcasted_iota(jnp.int32, (BLK, 1), 0)
    y = jnp.where(rowid < N_NODES, dis_ref[...] * x_ref[...], 0.0)
    y_ref[...] = y.astype(jnp.bfloat16)


def _scale_tc(x, dis):
    return pl.pallas_call(
        _scale_body,
        grid=(N_PAD // BLK,),
        in_specs=[
            pl.BlockSpec((BLK, D_HID), lambda i: (i, 0)),
            pl.BlockSpec((BLK, 1), lambda i: (i, 0)),
        ],
        out_specs=pl.BlockSpec((BLK, D_HID), lambda i: (i, 0)),
        out_shape=jax.ShapeDtypeStruct((N_PAD, D_HID), jnp.bfloat16),
    )(x, dis)


def _epi_body(s_ref, y_ref, dis_ref, b_ref, pw_ref, out_ref):
    ssum = (s_ref[0].astype(jnp.float32) + s_ref[1].astype(jnp.float32))
    o = dis_ref[...] * (ssum + y_ref[...].astype(jnp.float32)) + b_ref[...]
    out_ref[...] = jnp.where(o >= 0.0, o, pw_ref[...] * o)


BLK_E = 1000


def _epilogue_tc(s_parts, y, dis, b2, pw2):
    return pl.pallas_call(
        _epi_body,
        grid=(N_NODES // BLK_E,),
        in_specs=[
            pl.BlockSpec((2, BLK_E, D_HID), lambda i: (0, i, 0)),
            pl.BlockSpec((BLK_E, D_HID), lambda i: (i, 0)),
            pl.BlockSpec((BLK_E, 1), lambda i: (i, 0)),
            pl.BlockSpec((1, D_HID), lambda i: (0, 0)),
            pl.BlockSpec((1, D_HID), lambda i: (0, 0)),
        ],
        out_specs=pl.BlockSpec((BLK_E, D_HID), lambda i: (i, 0)),
        out_shape=jax.ShapeDtypeStruct((N_NODES, D_HID), jnp.float32),
    )(s_parts, y, dis, b2, pw2)


def kernel(span_reps, type_features, edge_index, edge_attr, emb_table, W, b,
           prelu_w):
    del edge_attr
    f32 = jnp.float32
    ei3 = edge_index.astype(jnp.int32).reshape(2, N_EDGES // CW, CW)
    npadrows = N_PAD - N_NODES
    pad3 = (SENTINEL + (jnp.arange(PAD_CPT * 32 * CW, dtype=jnp.int32)
                        % npadrows)).reshape(PAD_CPT * 32, CW)

    span_p = jnp.pad(span_reps, ((0, N_PAD - N_NODES), (0, 0)))
    types3 = jnp.pad(type_features.astype(jnp.int32),
                     (0, N_PAD - N_NODES)).reshape(N_PAD // BLK, 1, BLK)
    wt = W.astype(f32).T
    deg_parts = _deg_call()(ei3, pad3)
    x = _matmul_tc(span_p.astype(f32), types3, emb_table.astype(f32),
                   wt[:96], wt[96:])
    dis = lax.rsqrt(deg_parts[0] + deg_parts[1] + 1.0).reshape(N_PAD, 1)
    y = _scale_tc(x, dis)

    zeros_rows = jnp.zeros((N_PAD, D_HID), jnp.bfloat16)
    s_parts = _agg_call()(y, ei3, pad3, zeros_rows)

    return _epilogue_tc(s_parts, y, dis,
                        b.astype(f32).reshape(1, D_HID),
                        prelu_w.astype(f32).reshape(1, D_HID))
